# baseline (speedup 1.0000x reference)
import jax
import jax.numpy as jnp
from jax import lax
from jax.experimental import pallas as pl
from jax.experimental.pallas import tpu as pltpu
from jax.experimental.pallas import tpu_sc as plsc

B = 4096 * 200
D = 64
NW = 32
BPW = B // NW
C = 512
NCHUNK = BPW // C
SCALE = 8.0


def _body(idx_hbm, table_hbm, out_hbm, idx_v, rows_v, sem):
    wid = lax.axis_index("s") * 2 + lax.axis_index("c")
    base = wid * BPW

    def chunk(g, carry):
        off = base + g * C
        pltpu.sync_copy(idx_hbm.at[pl.ds(off, C)], idx_v)
        pltpu.async_copy(table_hbm.at[idx_v], rows_v, sem).wait()

        def row(r, carry2):
            for j in range(D // 16):
                sl = pl.ds(j * 16, 16)
                rows_v[r, sl] = rows_v[r, sl] * SCALE
            return carry2

        lax.fori_loop(0, C, row, 0)
        pltpu.sync_copy(rows_v, out_hbm.at[pl.ds(off, C)])
        return carry

    lax.fori_loop(0, NCHUNK, chunk, 0)


def kernel(x, table):
    xf = x.reshape(-1).astype(jnp.int32)
    out = pl.kernel(
        _body,
        mesh=plsc.VectorSubcoreMesh(core_axis_name="c", subcore_axis_name="s"),
        compiler_params=pltpu.CompilerParams(use_tc_tiling_on_sc=False),
        out_type=jax.ShapeDtypeStruct((B, D), jnp.float32),
        scratch_types=[
            pltpu.VMEM((C,), jnp.int32),
            pltpu.VMEM((C, D), jnp.float32),
            pltpu.SemaphoreType.DMA,
        ],
    )(xf, table)
    return out.reshape(x.shape[0], x.shape[1], D)


# speedup vs baseline: 1.1360x; 1.1360x over previous
import jax
import jax.numpy as jnp
from jax import lax
from jax.experimental import pallas as pl
from jax.experimental.pallas import tpu as pltpu
from jax.experimental.pallas import tpu_sc as plsc

B = 4096 * 200
D = 64
NW = 32
BPW = B // NW
C = 400
NCHUNK = BPW // C
NB = 4
LA = 2
SCALE = 8.0


def _body(idx_hbm, table_hbm, out_hbm, idx_v, rows_v, *sems):
    gsems = sems[0:NB]
    ssems = sems[NB:2 * NB]
    wid = lax.axis_index("s") * 2 + lax.axis_index("c")
    base = wid * BPW

    pltpu.sync_copy(idx_hbm.at[pl.ds(base, BPW)], idx_v)

    def issue_gather(g, b):
        pltpu.async_copy(table_hbm.at[idx_v.at[pl.ds(g * C, C)]],
                         rows_v.at[b], gsems[b])

    def wait_gather(g, b):
        pltpu.make_async_copy(table_hbm.at[idx_v.at[pl.ds(g * C, C)]],
                              rows_v.at[b], gsems[b]).wait()

    def issue_scatter(g, b):
        pltpu.async_copy(rows_v.at[b], out_hbm.at[pl.ds(base + g * C, C)],
                         ssems[b])

    def wait_scatter(b):
        pltpu.make_async_copy(rows_v.at[b], out_hbm.at[pl.ds(base, C)],
                              ssems[b]).wait()

    for g in range(LA):
        issue_gather(g, g % NB)

    @pl.loop(0, NCHUNK, step=NB)
    def _(t):
        for b in range(NB):
            g = t + b
            wait_gather(g, b)

            bb = (b + LA) % NB

            @pl.when(g + LA < NCHUNK)
            def _():
                @pl.when(g >= NB - LA)
                def _():
                    wait_scatter(bb)
                issue_gather(g + LA, bb)

            @plsc.parallel_loop(0, C, step=1, unroll=4)
            def _(r):
                for j in range(D // 16):
                    sl = pl.ds(j * 16, 16)
                    rows_v[b, r, sl] = rows_v[b, r, sl] * SCALE

            issue_scatter(g, b)

    for b in range(NB):
        wait_scatter(b)


def kernel(x, table):
    xf = x.reshape(-1).astype(jnp.int32)
    out = pl.kernel(
        _body,
        mesh=plsc.VectorSubcoreMesh(core_axis_name="c", subcore_axis_name="s"),
        compiler_params=pltpu.CompilerParams(use_tc_tiling_on_sc=False),
        out_type=jax.ShapeDtypeStruct((B, D), jnp.float32),
        scratch_types=[
            pltpu.VMEM((BPW,), jnp.int32),
            pltpu.VMEM((NB, C, D), jnp.float32),
        ] + [pltpu.SemaphoreType.DMA] * (2 * NB),
    )(xf, table)
    return out.reshape(x.shape[0], x.shape[1], D)
